# baseline (speedup 1.0000x reference)
import jax
import jax.numpy as jnp
from jax.experimental import pallas as pl
from jax.experimental.pallas import tpu as pltpu


def _round_up(x, m):
    return ((x + m - 1) // m) * m


def _mdn_body(x_ref, wsm_ref, bsm_ref, wpi_ref, bpi_ref,
              pi_ref, sigma_ref, mu_ref):
    x = x_ref[...].astype(jnp.bfloat16)
    go = sigma_ref.shape[-1]

    sm = jnp.dot(x, wsm_ref[...],
                 preferred_element_type=jnp.float32) + bsm_ref[...]
    sigma_ref[...] = jnp.exp(sm[:, :go]).astype(sigma_ref.dtype)
    mu_ref[...] = sm[:, go:].astype(mu_ref.dtype)

    logits = jnp.dot(x, wpi_ref[...],
                     preferred_element_type=jnp.float32) + bpi_ref[...]
    m = jnp.max(logits, axis=1, keepdims=True)
    e = jnp.exp(logits - m)
    pi_ref[...] = (e / jnp.sum(e, axis=1, keepdims=True)).astype(pi_ref.dtype)


def kernel(x, w_pi, b_pi, w_sigma, b_sigma, w_mu, b_mu):
    B, D = x.shape
    G = w_pi.shape[1]
    GO = w_sigma.shape[1]
    O = GO // G
    out_dtype = x.dtype

    w_sm = jnp.concatenate([w_sigma, w_mu], axis=1).astype(jnp.bfloat16)
    b_sm = jnp.concatenate([b_sigma, b_mu], axis=1)
    w_pi16 = w_pi.astype(jnp.bfloat16)

    TB = min(1024, max(8, _round_up(-(-B // 4), 8)))
    B_pad = _round_up(B, TB)
    if B_pad != B:
        x = jnp.pad(x, ((0, B_pad - B), (0, 0)))
    grid = (B_pad // TB,)

    pi_pad, sigma_pad, mu_pad = pl.pallas_call(
        _mdn_body,
        out_shape=(
            jax.ShapeDtypeStruct((B_pad, G), out_dtype),
            jax.ShapeDtypeStruct((B_pad, GO), out_dtype),
            jax.ShapeDtypeStruct((B_pad, GO), out_dtype),
        ),
        grid=grid,
        in_specs=[
            pl.BlockSpec((TB, D), lambda i: (i, 0)),
            pl.BlockSpec((D, 2 * GO), lambda i: (0, 0)),
            pl.BlockSpec((1, 2 * GO), lambda i: (0, 0)),
            pl.BlockSpec((D, G), lambda i: (0, 0)),
            pl.BlockSpec((1, G), lambda i: (0, 0)),
        ],
        out_specs=(
            pl.BlockSpec((TB, G), lambda i: (i, 0)),
            pl.BlockSpec((TB, GO), lambda i: (i, 0)),
            pl.BlockSpec((TB, GO), lambda i: (i, 0)),
        ),
        compiler_params=pltpu.CompilerParams(
            dimension_semantics=("parallel",),
            vmem_limit_bytes=64 * 1024 * 1024,
        ),
    )(x, w_sm, b_sm, w_pi16, b_pi)

    if B_pad != B:
        pi_pad = pi_pad[:B]
        sigma_pad = sigma_pad[:B]
        mu_pad = mu_pad[:B]
    return pi_pad, sigma_pad.reshape(B, G, O), mu_pad.reshape(B, G, O)


# speedup vs baseline: 1.0901x; 1.0901x over previous
import jax
import jax.numpy as jnp
from jax.experimental import pallas as pl
from jax.experimental.pallas import tpu as pltpu


def _round_up(x, m):
    return ((x + m - 1) // m) * m


def _mdn_body(x_ref, wsm_ref, bsm_ref, wpi_ref, bpi_ref,
              pi_ref, sigma_ref, mu_ref):
    x = x_ref[...].astype(jnp.bfloat16)
    go = sigma_ref.shape[-1]

    sm = jnp.dot(x, wsm_ref[...],
                 preferred_element_type=jnp.float32) + bsm_ref[...]
    sigma_ref[...] = jnp.exp(sm[:, :go]).astype(sigma_ref.dtype)
    mu_ref[...] = sm[:, go:].astype(mu_ref.dtype)

    logits = jnp.dot(x, wpi_ref[...],
                     preferred_element_type=jnp.float32) + bpi_ref[...]
    m = jnp.max(logits, axis=1, keepdims=True)
    e = jnp.exp(logits - m)
    pi_ref[...] = (e / jnp.sum(e, axis=1, keepdims=True)).astype(pi_ref.dtype)


def kernel(x, w_pi, b_pi, w_sigma, b_sigma, w_mu, b_mu):
    B, D = x.shape
    G = w_pi.shape[1]
    GO = w_sigma.shape[1]
    O = GO // G
    out_dtype = x.dtype

    w_sm = jnp.concatenate([w_sigma, w_mu], axis=1).astype(jnp.bfloat16)
    b_sm = jnp.concatenate([b_sigma, b_mu], axis=1)
    w_pi16 = w_pi.astype(jnp.bfloat16)

    TB = min(4096, max(8, _round_up(-(-B // 4), 8)))
    B_pad = _round_up(B, TB)
    if B_pad != B:
        x = jnp.pad(x, ((0, B_pad - B), (0, 0)))
    grid = (B_pad // TB,)

    pi_pad, sigma_pad, mu_pad = pl.pallas_call(
        _mdn_body,
        out_shape=(
            jax.ShapeDtypeStruct((B_pad, G), out_dtype),
            jax.ShapeDtypeStruct((B_pad, GO), out_dtype),
            jax.ShapeDtypeStruct((B_pad, GO), out_dtype),
        ),
        grid=grid,
        in_specs=[
            pl.BlockSpec((TB, D), lambda i: (i, 0)),
            pl.BlockSpec((D, 2 * GO), lambda i: (0, 0)),
            pl.BlockSpec((1, 2 * GO), lambda i: (0, 0)),
            pl.BlockSpec((D, G), lambda i: (0, 0)),
            pl.BlockSpec((1, G), lambda i: (0, 0)),
        ],
        out_specs=(
            pl.BlockSpec((TB, G), lambda i: (i, 0)),
            pl.BlockSpec((TB, GO), lambda i: (i, 0)),
            pl.BlockSpec((TB, GO), lambda i: (i, 0)),
        ),
        compiler_params=pltpu.CompilerParams(
            dimension_semantics=("parallel",),
            vmem_limit_bytes=64 * 1024 * 1024,
        ),
    )(x, w_sm, b_sm, w_pi16, b_pi)

    if B_pad != B:
        pi_pad = pi_pad[:B]
        sigma_pad = sigma_pad[:B]
        mu_pad = mu_pad[:B]
    return pi_pad, sigma_pad.reshape(B, G, O), mu_pad.reshape(B, G, O)
